# baseline (speedup 1.0000x reference)
import functools

import jax
import jax.numpy as jnp
from jax.experimental import pallas as pl
from jax.experimental.pallas import tpu as pltpu

TM = 1024


def _gcn_kernel(x_ref, w_ref, adj_ref, bias_ref, out_ref, support_ref):
    i = pl.program_id(1)

    @pl.when(i == 0)
    def _compute_support():
        support_ref[...] = jnp.dot(
            x_ref[0], w_ref[...], preferred_element_type=jnp.float32
        )

    @pl.when(i > 0)
    def _spmm_tile():
        out_ref[0] = (
            jnp.dot(adj_ref[0], support_ref[...], preferred_element_type=jnp.float32)
            + bias_ref[...]
        )


@jax.jit
def kernel(input, adj, W, b):
    B, N, D_IN = input.shape
    D_OUT = W.shape[1]
    grid = (B, N // TM + 1)

    return pl.pallas_call(
        _gcn_kernel,
        grid=grid,
        in_specs=[
            pl.BlockSpec((1, N, D_IN), lambda b_, i: (b_, 0, 0)),
            pl.BlockSpec((D_IN, D_OUT), lambda b_, i: (0, 0)),
            pl.BlockSpec((1, TM, N), lambda b_, i: (b_, jnp.maximum(i - 1, 0), 0)),
            pl.BlockSpec((1, D_OUT), lambda b_, i: (0, 0)),
        ],
        out_specs=pl.BlockSpec(
            (1, TM, D_OUT), lambda b_, i: (b_, jnp.maximum(i - 1, 0), 0)
        ),
        out_shape=jax.ShapeDtypeStruct((B, N, D_OUT), jnp.float32),
        scratch_shapes=[pltpu.VMEM((N, D_OUT), jnp.float32)],
        compiler_params=pltpu.CompilerParams(
            dimension_semantics=("arbitrary", "arbitrary"),
        ),
    )(input, W, adj, b.reshape(1, D_OUT))


# speedup vs baseline: 1.1089x; 1.1089x over previous
import functools

import jax
import jax.numpy as jnp
from jax.experimental import pallas as pl
from jax.experimental.pallas import tpu as pltpu

TM = 1024


def _gcn_kernel(x_ref, w_ref, adj_ref, bias_ref, out_ref, support_ref):
    i = pl.program_id(1)

    @pl.when(i == 0)
    def _compute_support():
        support_ref[...] = jnp.dot(
            x_ref[0], w_ref[...], preferred_element_type=jnp.float32
        )

    out_ref[0] = (
        jnp.dot(adj_ref[0], support_ref[...], preferred_element_type=jnp.float32)
        + bias_ref[...]
    )


@jax.jit
def kernel(input, adj, W, b):
    B, N, D_IN = input.shape
    D_OUT = W.shape[1]
    grid = (B, N // TM)

    return pl.pallas_call(
        _gcn_kernel,
        grid=grid,
        in_specs=[
            pl.BlockSpec((1, N, D_IN), lambda b_, i: (b_, 0, 0)),
            pl.BlockSpec((D_IN, D_OUT), lambda b_, i: (0, 0)),
            pl.BlockSpec((1, TM, N), lambda b_, i: (b_, i, 0)),
            pl.BlockSpec((1, D_OUT), lambda b_, i: (0, 0)),
        ],
        out_specs=pl.BlockSpec((1, TM, D_OUT), lambda b_, i: (b_, i, 0)),
        out_shape=jax.ShapeDtypeStruct((B, N, D_OUT), jnp.float32),
        scratch_shapes=[pltpu.VMEM((N, D_OUT), jnp.float32)],
        compiler_params=pltpu.CompilerParams(
            dimension_semantics=("arbitrary", "arbitrary"),
        ),
    )(input, W, adj, b.reshape(1, D_OUT))
